# Initial kernel scaffold; baseline (speedup 1.0000x reference)
import jax
import jax.numpy as jnp
from jax.experimental import pallas as pl


def kernel(l_data, edge_index, W0, b0, W1, b1, W2, b2):
    raise NotImplementedError("write your pallas kernel here")


# speedup vs baseline: 5.5834x; 5.5834x over previous
import functools

import jax
import jax.numpy as jnp
from jax import lax
from jax.experimental import pallas as pl
from jax.experimental.pallas import tpu as pltpu
from jax.experimental.pallas import tpu_sc as plsc

N = 100000
E = 1600000
HID = 32
LANES = 128
NC, NS = 2, 16
R_PAD = 12544
PAD_E = R_PAD * LANES
PAD_IDX = 100001
NP_ = 100096
ROWS_PER_TILE = R_PAD // NS
BLK = 56
NBLK = ROWS_PER_TILE // BLK
HALF = N // 2
ACC_ROWS = 50048
TRASH = HALF
CH32 = 184
NCH32 = 17
ACC1 = 50048
TRASH1 = 50016
DEGC = NP_ // NS

_mesh = plsc.VectorSubcoreMesh(core_axis_name="c", subcore_axis_name="s")
_f32 = jnp.float32


def _local_dst(didx, dloc, j, base, trash, limit):
    for k in range(LANES // 16):
        d = didx[j, pl.ds(k * 16, 16)] - base
        oob = (d < 0) | (d >= limit)
        dloc[j, pl.ds(k * 16, 16)] = jnp.where(oob, trash, d)


@functools.partial(
    pl.kernel,
    mesh=_mesh,
    compiler_params=pltpu.CompilerParams(use_tc_tiling_on_sc=False),
    out_type=jax.ShapeDtypeStruct((NC * NP_,), _f32),
    scratch_types=[
        pltpu.VMEM((BLK, LANES), jnp.int32),
        pltpu.VMEM((BLK, LANES), jnp.int32),
        pltpu.VMEM((BLK, LANES), jnp.int32),
        pltpu.VMEM((LANES,), _f32),
        pltpu.VMEM((DEGC,), _f32),
        pltpu.VMEM_SHARED((NP_,), _f32),
        pltpu.SemaphoreType.DMA,
    ],
)
def _deg_kernel(src2, dst2, ones_in, zeros1, out, sidx, didx, sel, ones_v,
                stage, acc, sem):
    c = lax.axis_index("c")
    s = lax.axis_index("s")
    pltpu.sync_copy(ones_in, ones_v)
    pltpu.sync_copy(zeros1, stage)
    pltpu.sync_copy(stage, acc.at[pl.ds(s * DEGC, DEGC)])
    plsc.subcore_barrier()
    row0 = s * ROWS_PER_TILE

    def blk_body(b, carry):
        r0 = row0 + b * BLK
        pltpu.sync_copy(src2.at[pl.ds(r0, BLK)], sidx)
        pltpu.sync_copy(dst2.at[pl.ds(r0, BLK)], didx)

        def row_body(j, carry2):
            for k in range(LANES // 16):
                sv = sidx[j, pl.ds(k * 16, 16)]
                dv = didx[j, pl.ds(k * 16, 16)]
                sel[j, pl.ds(k * 16, 16)] = sv * (1 - c) + dv * c
            pltpu.sync_copy(ones_v, acc.at[sel.at[j]], add=True)
            return carry2

        return lax.fori_loop(0, BLK, row_body, carry)

    lax.fori_loop(0, NBLK, blk_body, 0)
    plsc.subcore_barrier()
    pltpu.sync_copy(acc.at[pl.ds(s * DEGC, DEGC)], stage)
    pltpu.sync_copy(stage, out.at[pl.ds(c * NP_ + s * DEGC, DEGC)])


@functools.partial(
    pl.kernel,
    mesh=_mesh,
    compiler_params=pltpu.CompilerParams(use_tc_tiling_on_sc=False),
    out_type=jax.ShapeDtypeStruct((NC * ACC1,), _f32),
    scratch_types=[
        pltpu.VMEM((BLK, LANES), jnp.int32),
        pltpu.VMEM((BLK, LANES), jnp.int32),
        pltpu.VMEM((BLK, LANES), jnp.int32),
        pltpu.VMEM((LANES,), _f32),
        pltpu.VMEM((ACC1 // NS,), _f32),
        pltpu.VMEM_SHARED((ACC1,), _f32),
        pltpu.SemaphoreType.DMA,
    ],
)
def _mp1_kernel(x0tab, src2, dst2, zeros1, out, sidx, didx, dloc, vals,
                stage, acc, sem):
    c = lax.axis_index("c")
    s = lax.axis_index("s")
    base = c * HALF
    chunk = ACC1 // NS
    pltpu.sync_copy(zeros1, stage)
    pltpu.sync_copy(stage, acc.at[pl.ds(s * chunk, chunk)])
    plsc.subcore_barrier()
    row0 = s * ROWS_PER_TILE

    def blk_body(b, carry):
        r0 = row0 + b * BLK
        pltpu.sync_copy(src2.at[pl.ds(r0, BLK)], sidx)
        pltpu.sync_copy(dst2.at[pl.ds(r0, BLK)], didx)

        def row_body(j, carry2):
            _local_dst(didx, dloc, j, base, TRASH1, HALF)
            pltpu.async_copy(x0tab.at[sidx.at[j]], vals, sem).wait()
            pltpu.sync_copy(vals, acc.at[dloc.at[j]], add=True)
            return carry2

        return lax.fori_loop(0, BLK, row_body, carry)

    lax.fori_loop(0, NBLK, blk_body, 0)
    plsc.subcore_barrier()
    pltpu.sync_copy(acc.at[pl.ds(s * chunk, chunk)], stage)
    pltpu.sync_copy(stage, out.at[pl.ds(c * ACC1 + s * chunk, chunk)])


FH = HID // 2


@functools.partial(
    pl.kernel,
    mesh=_mesh,
    compiler_params=pltpu.CompilerParams(use_tc_tiling_on_sc=False),
    out_type=jax.ShapeDtypeStruct((4 * ACC_ROWS, FH), _f32),
    scratch_types=[
        pltpu.VMEM((BLK, LANES), jnp.int32),
        pltpu.VMEM((BLK, LANES), jnp.int32),
        pltpu.VMEM((BLK, LANES), jnp.int32),
        pltpu.VMEM((LANES, FH), _f32),
        pltpu.VMEM((CH32, FH), _f32),
        pltpu.VMEM_SHARED((ACC_ROWS, FH), _f32),
        pltpu.SemaphoreType.DMA,
    ],
)
def _mp32_kernel(ytab_lo, ytab_hi, src2, dst2, zeros2, out, sidx, didx, dloc,
                 rows, stage, acc, sem):
    c = lax.axis_index("c")
    s = lax.axis_index("s")
    base = c * HALF
    row0 = s * ROWS_PER_TILE
    for p, ytab in enumerate((ytab_lo, ytab_hi)):
        pltpu.sync_copy(zeros2, stage)
        for z in range(NCH32):
            pltpu.sync_copy(
                stage, acc.at[pl.ds(s * (NCH32 * CH32) + z * CH32, CH32)])
        plsc.subcore_barrier()

        def blk_body(b, carry):
            r0 = row0 + b * BLK
            pltpu.sync_copy(src2.at[pl.ds(r0, BLK)], sidx)
            pltpu.sync_copy(dst2.at[pl.ds(r0, BLK)], didx)

            def row_body(j, carry2):
                _local_dst(didx, dloc, j, base, TRASH, HALF)
                pltpu.async_copy(ytab.at[sidx.at[j]], rows, sem).wait()
                pltpu.sync_copy(rows, acc.at[dloc.at[j]], add=True)
                return carry2

            return lax.fori_loop(0, BLK, row_body, carry)

        lax.fori_loop(0, NBLK, blk_body, 0)
        plsc.subcore_barrier()
        for q in range(NCH32):
            r = s * (NCH32 * CH32) + q * CH32
            pltpu.sync_copy(acc.at[pl.ds(r, CH32)], stage)
            pltpu.sync_copy(
                stage, out.at[pl.ds((c * 2 + p) * ACC_ROWS + r, CH32)])
        plsc.subcore_barrier()


def _prep_body(od_ref, id_ref, ld_ref, ns_ref, nd_ref, x0_ref):
    od = od_ref[...]
    ind = id_ref[...]
    ns = jnp.where(od > 0, lax.rsqrt(jnp.maximum(od, 1.0)), 0.0)
    nd = jnp.where(ind > 0, lax.rsqrt(jnp.maximum(ind, 1.0)), 0.0)
    ns_ref[...] = ns
    nd_ref[...] = nd
    x0_ref[...] = ld_ref[...] * ns


_prep_call = pl.pallas_call(
    _prep_body,
    out_shape=[jax.ShapeDtypeStruct((3128, 32), _f32)] * 3,
)

_BL = 6256


def _l0_body(agg_ref, nd_ref, ns_ref, w0_ref, b0_ref, w1_ref,
             ylo_ref, yhi_ref):
    a = agg_ref[...] * nd_ref[...]
    h = jnp.maximum(a * w0_ref[0:1, :] + b0_ref[0:1, :], 0.0)
    y = jnp.dot(h * ns_ref[...], w1_ref[...], preferred_element_type=_f32)
    ylo_ref[...] = y[:, 0:FH]
    yhi_ref[...] = y[:, FH:HID]


_l0_call = pl.pallas_call(
    _l0_body,
    grid=(16,),
    in_specs=[
        pl.BlockSpec((_BL, 1), lambda i: (i, 0)),
        pl.BlockSpec((_BL, 1), lambda i: (i, 0)),
        pl.BlockSpec((_BL, 1), lambda i: (i, 0)),
        pl.BlockSpec((8, HID), lambda i: (0, 0)),
        pl.BlockSpec((8, HID), lambda i: (0, 0)),
        pl.BlockSpec((HID, HID), lambda i: (0, 0)),
    ],
    out_specs=[pl.BlockSpec((_BL, FH), lambda i: (i, 0)),
               pl.BlockSpec((_BL, FH), lambda i: (i, 0))],
    out_shape=[jax.ShapeDtypeStruct((NP_, FH), _f32),
               jax.ShapeDtypeStruct((NP_, FH), _f32)],
)


def _l1_body(agg_ref, nd_ref, ns_ref, b_ref, w_ref, ylo_ref, yhi_ref):
    h = jnp.maximum(agg_ref[...] * nd_ref[...] + b_ref[0:1, :], 0.0)
    y = jnp.dot(h * ns_ref[...], w_ref[...], preferred_element_type=_f32)
    ylo_ref[...] = y[:, 0:FH]
    yhi_ref[...] = y[:, FH:HID]


_l1_call = pl.pallas_call(
    _l1_body,
    grid=(16,),
    in_specs=[
        pl.BlockSpec((_BL, HID), lambda i: (i, 0)),
        pl.BlockSpec((_BL, 1), lambda i: (i, 0)),
        pl.BlockSpec((_BL, 1), lambda i: (i, 0)),
        pl.BlockSpec((8, HID), lambda i: (0, 0)),
        pl.BlockSpec((HID, HID), lambda i: (0, 0)),
    ],
    out_specs=[pl.BlockSpec((_BL, FH), lambda i: (i, 0)),
               pl.BlockSpec((_BL, FH), lambda i: (i, 0))],
    out_shape=[jax.ShapeDtypeStruct((NP_, FH), _f32),
               jax.ShapeDtypeStruct((NP_, FH), _f32)],
)

_BL2 = 5000


def _l2_body(agg_ref, nd_ref, b_ref, y_ref):
    y_ref[...] = agg_ref[...] * nd_ref[...] + b_ref[0:1, :]


_l2_call = pl.pallas_call(
    _l2_body,
    grid=(20,),
    in_specs=[
        pl.BlockSpec((_BL2, HID), lambda i: (i, 0)),
        pl.BlockSpec((_BL2, 1), lambda i: (i, 0)),
        pl.BlockSpec((8, HID), lambda i: (0, 0)),
    ],
    out_specs=pl.BlockSpec((_BL2, HID), lambda i: (i, 0)),
    out_shape=jax.ShapeDtypeStruct((N, HID), _f32),
)


def kernel(l_data, edge_index, W0, b0, W1, b1, W2, b2):
    ei = jnp.pad(edge_index, ((0, 0), (0, PAD_E - E)),
                 constant_values=PAD_IDX)
    src2 = ei[0].reshape(R_PAD, LANES)
    dst2 = ei[1].reshape(R_PAD, LANES)
    ones_in = jnp.ones((LANES,), _f32)
    zeros_deg = jnp.zeros((DEGC,), _f32)
    zeros_mp1 = jnp.zeros((ACC1 // NS,), _f32)
    zeros_mp32 = jnp.zeros((CH32, HID // 2), _f32)

    deg = _deg_kernel(src2, dst2, ones_in, zeros_deg)
    od2 = deg[:NP_].reshape(3128, 32)
    id2 = deg[NP_:].reshape(3128, 32)
    ld2 = jnp.pad(l_data[:, 0], (0, NP_ - N)).reshape(3128, 32)
    ns2, nd2, x02 = _prep_call(od2, id2, ld2)
    x0 = x02.reshape(NP_)
    nsc = ns2.reshape(NP_, 1)
    ndc = nd2.reshape(NP_, 1)

    a0 = _mp1_kernel(x0, src2, dst2, zeros_mp1)
    agg0 = jnp.concatenate(
        [a0[:HALF], a0[ACC1:ACC1 + HALF], jnp.zeros((NP_ - N,), _f32)]
    ).reshape(NP_, 1)

    w0b = jnp.broadcast_to(W0, (8, HID))
    b0b = jnp.broadcast_to(b0[None, :], (8, HID))
    b1b = jnp.broadcast_to(b1[None, :], (8, HID))
    b2b = jnp.broadcast_to(b2[None, :], (8, HID))

    pad_rows = jnp.zeros((NP_ - N, HID), _f32)

    def _assemble(o):
        top = jnp.concatenate(
            [o[:HALF], o[ACC_ROWS:ACC_ROWS + HALF]], axis=1)
        bot = jnp.concatenate(
            [o[2 * ACC_ROWS:2 * ACC_ROWS + HALF],
             o[3 * ACC_ROWS:3 * ACC_ROWS + HALF]], axis=1)
        return jnp.concatenate([top, bot, pad_rows], axis=0)

    y1lo, y1hi = _l0_call(agg0, ndc, nsc, w0b, b0b, W1)
    agg1 = _assemble(_mp32_kernel(y1lo, y1hi, src2, dst2, zeros_mp32))
    y2lo, y2hi = _l1_call(agg1, ndc, nsc, b1b, W2)
    agg2 = _assemble(_mp32_kernel(y2lo, y2hi, src2, dst2, zeros_mp32))
    return _l2_call(agg2[:N], ndc[:N], b2b)
